# baseline (speedup 1.0000x reference)
import functools

import numpy as np
import jax
import jax.numpy as jnp
from jax import lax
from jax.experimental import pallas as pl
from jax.experimental.pallas import tpu as pltpu
from jax.experimental.pallas import tpu_sc as plsc

B = 128
N = 32768
ROWS = 256
RB = 128
GRID = ROWS // RB
NC = 1
GROUPS = 8


def _stats_body(x_ref, activ_ref, y_ref, cdf_ref, t_ref, ac_scr):
    j = pl.program_id(0)
    xb = x_ref[...]
    y_ref[...] = xb
    colsum = jnp.sum(jnp.abs(xb), axis=0)
    a = 0.97 * activ_ref[...] + (0.03 / B) * colsum
    ac_scr[pl.ds(j * RB, RB), :] = jnp.exp(-5.0 * a)

    @pl.when(j == GRID - 1)
    def _():
        acm = ac_scr[...]
        li = lax.broadcasted_iota(jnp.int32, (ROWS, 128), 1)
        within = acm
        for k in (1, 2, 4, 8, 16, 32, 64):
            within = within + jnp.where(
                li >= k, pltpu.roll(within, k, 1), 0.0)
        rowtot = lax.slice(within, (0, 127), (ROWS, 128))
        si = lax.broadcasted_iota(jnp.int32, (ROWS, 1), 0)
        oincl = rowtot
        for k in (1, 2, 4, 8, 16, 32, 64, 128):
            oincl = oincl + jnp.where(
                si >= k, pltpu.roll(oincl, k, 0), 0.0)
        stot = lax.slice(oincl, (ROWS - 1, 0), (ROWS, 1))
        ac0 = lax.slice(acm, (0, 0), (1, 1))
        cdf_ref[...] = within + (oincl - rowtot)

        u32 = jnp.uint32
        i0 = lax.broadcasted_iota(jnp.int32, (8, 128), 0)
        i1 = lax.broadcasted_iota(jnp.int32, (8, 128), 1)
        cnt = (i0 * 128 + i1).astype(u32)
        x0 = cnt
        x1 = cnt + u32(1024)
        k0 = u32(0)
        k1 = u32(42)
        k2 = u32(np.uint32(0 ^ 42 ^ 0x1BD11BDA))
        ks = (k0, k1, k2)
        x0 = x0 + k0
        x1 = x1 + k1
        rot = ((13, 15, 26, 6), (17, 29, 16, 24))
        for g in range(5):
            for r in rot[g % 2]:
                x0 = x0 + x1
                x1 = (x1 << u32(r)) | (x1 >> u32(32 - r))
                x1 = x1 ^ x0
            x0 = x0 + ks[(g + 1) % 3]
            x1 = x1 + ks[(g + 2) % 3] + u32(g + 1)
        bits = lax.slice(x0, (0, 0), (2, 128))
        uu = lax.bitcast_convert_type(
            (bits >> u32(9)) | u32(0x3F800000), jnp.float32) - 1.0
        u_pos = lax.slice(uu, (0, 0), (1, 128))
        u_branch = lax.slice(uu, (1, 0), (2, 128))
        total = 4001.0 * stot - ac0
        zero_branch = u_branch * total < 4000.0 * stot
        t_pos = ac0 + u_pos * (stot - ac0)
        t = jnp.where(zero_branch, -1.0, t_pos)
        cnt = jnp.sum((oincl <= t).astype(jnp.int32), axis=0, keepdims=True)
        b = jnp.minimum(cnt, ROWS - 1)
        t_ref[...] = jnp.concatenate(
            [t, lax.bitcast_convert_type(b, jnp.float32)], axis=0)


def _stats(x3, activ2):
    return pl.pallas_call(
        _stats_body,
        grid=(GRID,),
        in_specs=[
            pl.BlockSpec((B, RB, 128), lambda j: (0, j, 0)),
            pl.BlockSpec((RB, 128), lambda j: (j, 0)),
        ],
        out_specs=[
            pl.BlockSpec((B, RB, 128), lambda j: (0, j, 0)),
            pl.BlockSpec((ROWS, 128), lambda j: (0, 0)),
            pl.BlockSpec((2, 128), lambda j: (0, 0)),
        ],
        out_shape=[
            jax.ShapeDtypeStruct((B, ROWS, 128), jnp.float32),
            jax.ShapeDtypeStruct((ROWS, 128), jnp.float32),
            jax.ShapeDtypeStruct((2, 128), jnp.float32),
        ],
        scratch_shapes=[pltpu.VMEM((ROWS, 128), jnp.float32)],
    )(x3, activ2)


def _sc_sample_body(cdf_hbm, tb_hbm, r_hbm, tb_v, rows_v, r_v, sem):
    wid = lax.axis_index("s") * NC + lax.axis_index("c")

    @pl.when(wid < GROUPS)
    def _():
        base = wid * 16
        pltpu.sync_copy(tb_hbm, tb_v)
        t = tb_v[pl.ds(base, 16)]
        row = plsc.bitcast(tb_v[pl.ds(128 + base, 16)], jnp.int32)
        lane = lax.iota(jnp.int32, 16)
        pltpu.async_copy(cdf_hbm.at[row], rows_v, sem).wait()
        lo2 = jnp.zeros((16,), jnp.int32)
        hi2 = jnp.full((16,), 128, jnp.int32)
        for _ in range(8):
            mid = (lo2 + hi2) >> 1
            v = plsc.load_gather(rows_v, [lane, jnp.minimum(mid, 127)])
            pred = t < v
            hi2 = jnp.where(pred, mid, hi2)
            lo2 = jnp.where(pred, lo2, mid + 1)
        col = jnp.minimum(lo2, 127)
        r = row * 128 + col
        r_v[...] = jnp.where(r > 0, (base + lane) * N + r, 0)
        pltpu.sync_copy(r_v, r_hbm.at[pl.ds(base, 16)])


def _sc_sample(cdf2d, tb):
    mesh = plsc.VectorSubcoreMesh(core_axis_name="c", subcore_axis_name="s",
                                  num_cores=NC)
    k = functools.partial(
        pl.kernel,
        out_type=jax.ShapeDtypeStruct((B,), jnp.int32),
        mesh=mesh,
        compiler_params=pltpu.CompilerParams(needs_layout_passes=False),
        scratch_types=[
            pltpu.VMEM((2 * 128,), jnp.float32),
            pltpu.VMEM((16, 128), jnp.float32),
            pltpu.VMEM((16,), jnp.int32),
            pltpu.SemaphoreType.DMA,
        ],
    )(_sc_sample_body)
    return k(cdf2d, tb)


def _scatter_body(rw_ref, y_in_ref, std_ref, y_out_ref, buf, sem):
    for i in range(B):
        @pl.when(rw_ref[i] > 0)
        def _(i=i):
            idx = rw_ref[i]
            row = idx >> 15
            blk = (idx & 32767) >> 7
            col = idx & 127
            src = y_in_ref.at[pl.ds(row, 1), pl.ds(blk, 1)]
            pltpu.make_async_copy(src, buf, sem).start()
            pltpu.make_async_copy(src, buf, sem).wait()
            i3 = lax.broadcasted_iota(jnp.int32, (1, 1, 1, 128), 3)
            buf[...] = buf[...] + jnp.where(i3 == col, std_ref[...], 0.0)
            dst = y_out_ref.at[pl.ds(row, 1), pl.ds(blk, 1)]
            pltpu.make_async_copy(buf, dst, sem).start()
            pltpu.make_async_copy(buf, dst, sem).wait()


def _scatter(rw, y4, std4):
    return pl.pallas_call(
        _scatter_body,
        grid_spec=pltpu.PrefetchScalarGridSpec(
            num_scalar_prefetch=1,
            grid=(1,),
            in_specs=[
                pl.BlockSpec(memory_space=pl.ANY),
                pl.BlockSpec((1, 1, 1, 1), lambda g, rw_ref: (0, 0, 0, 0)),
            ],
            out_specs=pl.BlockSpec(memory_space=pl.ANY),
            scratch_shapes=[
                pltpu.VMEM((1, 1, 1, 128), jnp.float32),
                pltpu.SemaphoreType.DMA,
            ],
        ),
        out_shape=jax.ShapeDtypeStruct((B, ROWS, 1, 128), jnp.float32),
        input_output_aliases={1: 0},
    )(rw, y4, std4)


def kernel(x, std, activ):
    x3 = x.reshape(B, ROWS, 128)
    activ2 = activ.reshape(ROWS, 128)
    y3, cdf2d, tb = _stats(x3, activ2)
    rw = _sc_sample(cdf2d, tb.reshape(2 * 128))
    y4 = _scatter(rw, y3.reshape(B, ROWS, 1, 128),
                  std.reshape(1, 1, 1, 1))
    return y4.reshape(B, 1, N)


# speedup vs baseline: 1.1718x; 1.1718x over previous
import functools

import numpy as np
import jax
import jax.numpy as jnp
from jax import lax
from jax.experimental import pallas as pl
from jax.experimental.pallas import tpu as pltpu
from jax.experimental.pallas import tpu_sc as plsc

B = 128
N = 32768
ROWS = 256
RB = 128
GRID = ROWS // RB
NC = 1
GROUPS = 8


def _stats_body(x_ref, activ_ref, y_ref, cdf_ref, t_ref, ac_scr):
    j = pl.program_id(0)
    xb = x_ref[...]
    y_ref[...] = xb
    colsum = jnp.sum(jnp.abs(xb), axis=0)
    a = 0.97 * activ_ref[...] + (0.03 / B) * colsum
    ac_scr[pl.ds(j * RB, RB), :] = jnp.exp(-5.0 * a)

    @pl.when(j == GRID - 1)
    def _():
        acm = ac_scr[...]
        li = lax.broadcasted_iota(jnp.int32, (ROWS, 128), 1)
        within = acm
        for k in (1, 2, 4, 8, 16, 32, 64):
            within = within + jnp.where(
                li >= k, pltpu.roll(within, k, 1), 0.0)
        rowtot = lax.slice(within, (0, 127), (ROWS, 128))
        si = lax.broadcasted_iota(jnp.int32, (ROWS, 1), 0)
        oincl = rowtot
        for k in (1, 2, 4, 8, 16, 32, 64, 128):
            oincl = oincl + jnp.where(
                si >= k, pltpu.roll(oincl, k, 0), 0.0)
        stot = lax.slice(oincl, (ROWS - 1, 0), (ROWS, 1))
        ac0 = lax.slice(acm, (0, 0), (1, 1))
        cdf_ref[...] = within + (oincl - rowtot)

        u32 = jnp.uint32
        i0 = lax.broadcasted_iota(jnp.int32, (8, 128), 0)
        i1 = lax.broadcasted_iota(jnp.int32, (8, 128), 1)
        cnt = (i0 * 128 + i1).astype(u32)
        x0 = cnt
        x1 = cnt + u32(1024)
        k0 = u32(0)
        k1 = u32(42)
        k2 = u32(np.uint32(0 ^ 42 ^ 0x1BD11BDA))
        ks = (k0, k1, k2)
        x0 = x0 + k0
        x1 = x1 + k1
        rot = ((13, 15, 26, 6), (17, 29, 16, 24))
        for g in range(5):
            for r in rot[g % 2]:
                x0 = x0 + x1
                x1 = (x1 << u32(r)) | (x1 >> u32(32 - r))
                x1 = x1 ^ x0
            x0 = x0 + ks[(g + 1) % 3]
            x1 = x1 + ks[(g + 2) % 3] + u32(g + 1)
        bits = lax.slice(x0, (0, 0), (2, 128))
        uu = lax.bitcast_convert_type(
            (bits >> u32(9)) | u32(0x3F800000), jnp.float32) - 1.0
        u_pos = lax.slice(uu, (0, 0), (1, 128))
        u_branch = lax.slice(uu, (1, 0), (2, 128))
        total = 4001.0 * stot - ac0
        zero_branch = u_branch * total < 4000.0 * stot
        t_pos = ac0 + u_pos * (stot - ac0)
        t = jnp.where(zero_branch, -1.0, t_pos)
        cnt = jnp.sum((oincl <= t).astype(jnp.int32), axis=0, keepdims=True)
        b = jnp.minimum(cnt, ROWS - 1)
        t_ref[...] = jnp.concatenate(
            [t, lax.bitcast_convert_type(b, jnp.float32)], axis=0)


def _stats(x3, activ2):
    return pl.pallas_call(
        _stats_body,
        grid=(GRID,),
        in_specs=[
            pl.BlockSpec((B, RB, 128), lambda j: (0, j, 0)),
            pl.BlockSpec((RB, 128), lambda j: (j, 0)),
        ],
        out_specs=[
            pl.BlockSpec((B, RB, 128), lambda j: (0, j, 0)),
            pl.BlockSpec((ROWS, 128), lambda j: (0, 0)),
            pl.BlockSpec((2, 128), lambda j: (0, 0)),
        ],
        out_shape=[
            jax.ShapeDtypeStruct((B, ROWS, 128), jnp.float32),
            jax.ShapeDtypeStruct((ROWS, 128), jnp.float32),
            jax.ShapeDtypeStruct((2, 128), jnp.float32),
        ],
        scratch_shapes=[pltpu.VMEM((ROWS, 128), jnp.float32)],
    )(x3, activ2)


def _sc_sample_body(cdf_hbm, tb_hbm, r_hbm, tb_v, rows_v, r_v, sem):
    wid = lax.axis_index("s") * NC + lax.axis_index("c")

    @pl.when(wid < GROUPS)
    def _():
        base = wid * 16
        pltpu.sync_copy(tb_hbm, tb_v)
        t = tb_v[pl.ds(base, 16)]
        r_v[...] = jnp.zeros((16,), jnp.int32)

        @pl.when(jnp.any(t >= 0.0))
        def _():
            row = plsc.bitcast(tb_v[pl.ds(128 + base, 16)], jnp.int32)
            lane = lax.iota(jnp.int32, 16)
            pltpu.async_copy(cdf_hbm.at[row], rows_v, sem).wait()
            lo2 = jnp.zeros((16,), jnp.int32)
            hi2 = jnp.full((16,), 128, jnp.int32)
            for _ in range(8):
                mid = (lo2 + hi2) >> 1
                v = plsc.load_gather(rows_v, [lane, jnp.minimum(mid, 127)])
                pred = t < v
                hi2 = jnp.where(pred, mid, hi2)
                lo2 = jnp.where(pred, lo2, mid + 1)
            col = jnp.minimum(lo2, 127)
            r = row * 128 + col
            r_v[...] = jnp.where(r > 0, (base + lane) * N + r, 0)

        pltpu.sync_copy(r_v, r_hbm.at[pl.ds(base, 16)])


def _sc_sample(cdf2d, tb):
    mesh = plsc.VectorSubcoreMesh(core_axis_name="c", subcore_axis_name="s",
                                  num_cores=NC)
    k = functools.partial(
        pl.kernel,
        out_type=jax.ShapeDtypeStruct((B,), jnp.int32),
        mesh=mesh,
        compiler_params=pltpu.CompilerParams(needs_layout_passes=False),
        scratch_types=[
            pltpu.VMEM((2 * 128,), jnp.float32),
            pltpu.VMEM((16, 128), jnp.float32),
            pltpu.VMEM((16,), jnp.int32),
            pltpu.SemaphoreType.DMA,
        ],
    )(_sc_sample_body)
    return k(cdf2d, tb)


def _scatter_body(rw_ref, y_in_ref, std_ref, y_out_ref, buf, sem):
    for i in range(B):
        @pl.when(rw_ref[i] > 0)
        def _(i=i):
            idx = rw_ref[i]
            row = idx >> 15
            blk = (idx & 32767) >> 7
            col = idx & 127
            src = y_in_ref.at[pl.ds(row, 1), pl.ds(blk, 1)]
            pltpu.make_async_copy(src, buf, sem).start()
            pltpu.make_async_copy(src, buf, sem).wait()
            i3 = lax.broadcasted_iota(jnp.int32, (1, 1, 1, 128), 3)
            buf[...] = buf[...] + jnp.where(i3 == col, std_ref[...], 0.0)
            dst = y_out_ref.at[pl.ds(row, 1), pl.ds(blk, 1)]
            pltpu.make_async_copy(buf, dst, sem).start()
            pltpu.make_async_copy(buf, dst, sem).wait()


def _scatter(rw, y4, std4):
    return pl.pallas_call(
        _scatter_body,
        grid_spec=pltpu.PrefetchScalarGridSpec(
            num_scalar_prefetch=1,
            grid=(1,),
            in_specs=[
                pl.BlockSpec(memory_space=pl.ANY),
                pl.BlockSpec((1, 1, 1, 1), lambda g, rw_ref: (0, 0, 0, 0)),
            ],
            out_specs=pl.BlockSpec(memory_space=pl.ANY),
            scratch_shapes=[
                pltpu.VMEM((1, 1, 1, 128), jnp.float32),
                pltpu.SemaphoreType.DMA,
            ],
        ),
        out_shape=jax.ShapeDtypeStruct((B, ROWS, 1, 128), jnp.float32),
        input_output_aliases={1: 0},
    )(rw, y4, std4)


def kernel(x, std, activ):
    x3 = x.reshape(B, ROWS, 128)
    activ2 = activ.reshape(ROWS, 128)
    y3, cdf2d, tb = _stats(x3, activ2)
    rw = _sc_sample(cdf2d, tb.reshape(2 * 128))
    y4 = _scatter(rw, y3.reshape(B, ROWS, 1, 128),
                  std.reshape(1, 1, 1, 1))
    return y4.reshape(B, 1, N)
